# baseline (speedup 1.0000x reference)
import functools

import jax
import jax.numpy as jnp
from jax import lax
from jax.experimental import pallas as pl
from jax.experimental.pallas import tpu as pltpu
from jax.experimental.pallas import tpu_sc as plsc


CB = 8192
CHUNK = 128
NB = 2


def _tc_body(wt_ref, out_ref):
    eye = jnp.eye(64, dtype=jnp.float32)
    t = lax.dot_general(wt_ref[...], eye, (((0,), (0,)), ((), ())),
                        preferred_element_type=jnp.float32)
    out_ref[...] = jnp.concatenate([t, t], axis=1)


@functools.lru_cache(maxsize=None)
def _build_tc(V, D):
    grid = (V + CB - 1) // CB
    return pl.pallas_call(
        _tc_body,
        grid=(grid,),
        in_specs=[pl.BlockSpec((D, CB), lambda i: (0, i))],
        out_specs=pl.BlockSpec((CB, 2 * D), lambda i: (i, 0)),
        out_shape=jax.ShapeDtypeStruct((V, 2 * D), jnp.float32),
    )


@functools.lru_cache(maxsize=None)
def _build_sc(B0, S, V, NC, NS):
    NW = NC * NS
    n_chunks_total = B0 * S // CHUNK
    n_chunks = n_chunks_total // NW
    n_pass = n_chunks // NB
    assert n_chunks % NB == 0 and n_pass % 2 == 0
    NBT = B0 // CHUNK

    mesh = plsc.VectorSubcoreMesh(
        core_axis_name="c", subcore_axis_name="s",
        num_cores=NC, num_subcores=NS,
    )

    @functools.partial(
        pl.kernel,
        out_type=jax.ShapeDtypeStruct((S, 8, NBT, 8 * CHUNK), jnp.float32),
        mesh=mesh,
        scratch_types=[
            pltpu.VMEM((2, NB, CHUNK), jnp.int32),
            pltpu.VMEM((NB, CHUNK, 128), jnp.float32),
            pltpu.VMEM((NB, 8 * 8 * CHUNK), jnp.float32),
            pltpu.SemaphoreType.DMA,
            pltpu.SemaphoreType.DMA,
            pltpu.SemaphoreType.DMA,
        ],
        compiler_params=pltpu.CompilerParams(
            use_tc_tiling_on_sc=False, needs_layout_passes=False),
    )
    def sc_kernel(idx_hbm, table_hbm, out_hbm, idx_v, rows_v, stage_v,
                  isem, gsem, ssem):
        wid = lax.axis_index("s") * NC + lax.axis_index("c")
        c0 = wid * n_chunks

        def idx_fetch(g, p):
            pltpu.async_copy(
                idx_hbm.at[pl.ds(c0 + g * NB, NB)], idx_v.at[p], isem)

        def extract(b):
            lane = lax.iota(jnp.int32, 16)

            @plsc.parallel_loop(0, CHUNK // 16)
            def body_tb(tb):
                r0 = 16 * tb
                row_idx = r0 + lane
                for db in range(4):
                    for j in range(16):
                        skew = (lane + j) & 15
                        d_vec = 16 * db + skew
                        v = plsc.load_gather(
                            rows_v.at[b], [row_idx, d_vec])
                        plsc.store_scatter(
                            stage_v.at[b],
                            [d_vec * CHUNK + row_idx], v)

        def drain_stores():
            for _ in range(NB * 8):
                pltpu.make_async_copy(
                    stage_v.at[0, pl.ds(0, CHUNK * 8)],
                    out_hbm.at[0, 0, 0], ssem).wait()

        def one_pass(g2, q):
            g = 2 * g2 + q
            pltpu.make_async_copy(
                idx_hbm.at[pl.ds(0, NB)], idx_v.at[q], isem).wait()

            @pl.when(g >= 1)
            def _():
                drain_stores()

            for b in range(NB):
                pltpu.async_copy(
                    table_hbm.at[idx_v.at[q, b]], rows_v.at[b], gsem)

            @pl.when(g + 1 < n_pass)
            def _():
                idx_fetch(g + 1, 1 - q)

            for b in range(NB):
                pltpu.make_async_copy(
                    table_hbm.at[idx_v.at[q, b]], rows_v.at[b], gsem).wait()
                extract(b)
                c = c0 + g * NB + b
                s = c // NBT
                bt = lax.rem(c, NBT)
                for dt in range(8):
                    pltpu.async_copy(
                        stage_v.at[b, pl.ds(dt * CHUNK * 8, CHUNK * 8)],
                        out_hbm.at[s, dt, bt], ssem)

        idx_fetch(0, 0)

        def outer(g2, carry):
            one_pass(g2, 0)
            one_pass(g2, 1)
            return carry

        lax.fori_loop(0, n_pass // 2, outer, 0)
        drain_stores()

    return sc_kernel


def kernel(token_ids, weight):
    B0, S = token_ids.shape
    V, D = weight.shape
    B = B0 * S
    info = plsc.get_sparse_core_info()
    table = _build_tc(V, D)(jnp.transpose(weight))
    idx2d = jnp.transpose(token_ids).reshape(B // CHUNK, CHUNK)
    idx2d = idx2d.astype(jnp.int32)
    z = _build_sc(B0, S, V, info.num_cores, info.num_subcores)(idx2d, table)
    z = z.reshape(S, 8, B0 // CHUNK, 8, CHUNK)
    return z.transpose(2, 4, 0, 1, 3).reshape(B0, S, D)


# speedup vs baseline: 1.3758x; 1.3758x over previous
import functools

import jax
import jax.numpy as jnp
from jax import lax
from jax.experimental import pallas as pl
from jax.experimental.pallas import tpu as pltpu
from jax.experimental.pallas import tpu_sc as plsc


CB = 8192
CHUNK = 128
NB = 2


def _tc_body(wt_ref, out_ref):
    t = jnp.transpose(wt_ref[...])
    out_ref[...] = jnp.concatenate([t, t], axis=1)


@functools.lru_cache(maxsize=None)
def _build_tc(V, D):
    grid = (V + CB - 1) // CB
    return pl.pallas_call(
        _tc_body,
        grid=(grid,),
        in_specs=[pl.BlockSpec((D, CB), lambda i: (0, i))],
        out_specs=pl.BlockSpec((CB, 2 * D), lambda i: (i, 0)),
        out_shape=jax.ShapeDtypeStruct((V, 2 * D), jnp.float32),
    )


@functools.lru_cache(maxsize=None)
def _build_sc(B0, S, V, NC, NS):
    NW = NC * NS
    n_chunks_total = B0 * S // CHUNK
    n_chunks = n_chunks_total // NW
    n_pass = n_chunks // NB
    assert n_chunks % NB == 0 and n_pass % 2 == 0
    NBT = B0 // CHUNK

    mesh = plsc.VectorSubcoreMesh(
        core_axis_name="c", subcore_axis_name="s",
        num_cores=NC, num_subcores=NS,
    )

    @functools.partial(
        pl.kernel,
        out_type=jax.ShapeDtypeStruct((S, 8, NBT, 8 * CHUNK), jnp.float32),
        mesh=mesh,
        scratch_types=[
            pltpu.VMEM((2, NB, CHUNK), jnp.int32),
            pltpu.VMEM((NB, CHUNK, 128), jnp.float32),
            pltpu.VMEM((NB, 8 * 8 * CHUNK), jnp.float32),
            pltpu.SemaphoreType.DMA,
            pltpu.SemaphoreType.DMA,
            pltpu.SemaphoreType.DMA,
        ],
        compiler_params=pltpu.CompilerParams(
            use_tc_tiling_on_sc=False, needs_layout_passes=False),
    )
    def sc_kernel(idx_hbm, table_hbm, out_hbm, idx_v, rows_v, stage_v,
                  isem, gsem, ssem):
        wid = lax.axis_index("s") * NC + lax.axis_index("c")
        c0 = wid * n_chunks

        def idx_fetch(g, p):
            pltpu.async_copy(
                idx_hbm.at[pl.ds(c0 + g * NB, NB)], idx_v.at[p], isem)

        def extract(b):
            lane = lax.iota(jnp.int32, 16)

            @plsc.parallel_loop(0, CHUNK // 16)
            def body_tb(tb):
                r0 = 16 * tb
                row_idx = r0 + lane
                for db in range(4):
                    for jg in range(2):
                        vs = []
                        for j8 in range(8):
                            j = 8 * jg + j8
                            skew = (lane + j) & 15
                            d_vec = 16 * db + skew
                            vs.append((d_vec, plsc.load_gather(
                                rows_v.at[b], [row_idx, d_vec])))
                        for d_vec, v in vs:
                            plsc.store_scatter(
                                stage_v.at[b],
                                [d_vec * CHUNK + row_idx], v)

        def drain_stores():
            for _ in range(NB * 8):
                pltpu.make_async_copy(
                    stage_v.at[0, pl.ds(0, CHUNK * 8)],
                    out_hbm.at[0, 0, 0], ssem).wait()

        def one_pass(g2, q):
            g = 2 * g2 + q
            pltpu.make_async_copy(
                idx_hbm.at[pl.ds(0, NB)], idx_v.at[q], isem).wait()

            @pl.when(g >= 1)
            def _():
                drain_stores()

            for b in range(NB):
                pltpu.async_copy(
                    table_hbm.at[idx_v.at[q, b]], rows_v.at[b], gsem)

            @pl.when(g + 1 < n_pass)
            def _():
                idx_fetch(g + 1, 1 - q)

            for b in range(NB):
                pltpu.make_async_copy(
                    table_hbm.at[idx_v.at[q, b]], rows_v.at[b], gsem).wait()
                extract(b)
                c = c0 + g * NB + b
                s = c // NBT
                bt = lax.rem(c, NBT)
                for dt in range(8):
                    pltpu.async_copy(
                        stage_v.at[b, pl.ds(dt * CHUNK * 8, CHUNK * 8)],
                        out_hbm.at[s, dt, bt], ssem)

        idx_fetch(0, 0)

        def outer(g2, carry):
            one_pass(g2, 0)
            one_pass(g2, 1)
            return carry

        lax.fori_loop(0, n_pass // 2, outer, 0)
        drain_stores()

    return sc_kernel


def kernel(token_ids, weight):
    B0, S = token_ids.shape
    V, D = weight.shape
    B = B0 * S
    info = plsc.get_sparse_core_info()
    table = _build_tc(V, D)(jnp.transpose(weight))
    idx2d = jnp.transpose(token_ids).reshape(B // CHUNK, CHUNK)
    idx2d = idx2d.astype(jnp.int32)
    z = _build_sc(B0, S, V, info.num_cores, info.num_subcores)(idx2d, table)
    z = z.reshape(S, 8, B0 // CHUNK, 8, CHUNK)
    return z.transpose(2, 4, 0, 1, 3).reshape(B0, S, D)


# speedup vs baseline: 1.5389x; 1.1186x over previous
import functools

import jax
import jax.numpy as jnp
from jax import lax
from jax.experimental import pallas as pl
from jax.experimental.pallas import tpu as pltpu
from jax.experimental.pallas import tpu_sc as plsc


CB = 8192
CHUNK = 128
NB = 2


def _tc_body(wtl_ref, wtr_ref, out_ref):
    tl = jnp.transpose(wtl_ref[...])
    tr = jnp.transpose(wtr_ref[...])
    out_ref[...] = jnp.concatenate([tl, tr], axis=1)


@functools.lru_cache(maxsize=None)
def _build_tc(V, D):
    grid = (V + 2 * CB - 1) // (2 * CB)
    nblk = (V + CB - 1) // CB
    return pl.pallas_call(
        _tc_body,
        grid=(grid,),
        in_specs=[pl.BlockSpec((D, CB), lambda i: (0, i)),
                  pl.BlockSpec(
                      (D, CB),
                      lambda i: (0, jnp.minimum(62 + i, nblk - 1)))],
        out_specs=pl.BlockSpec((CB, 2 * D), lambda i: (i, 0)),
        out_shape=jax.ShapeDtypeStruct((grid * CB, 2 * D), jnp.float32),
    )


@functools.lru_cache(maxsize=None)
def _build_sc(B0, S, V, NC, NS):
    NW = NC * NS
    n_chunks_total = B0 * S // CHUNK
    n_chunks = n_chunks_total // NW
    n_pass = n_chunks // NB
    assert n_chunks % NB == 0 and n_pass % 2 == 0
    NBT = B0 // CHUNK

    mesh = plsc.VectorSubcoreMesh(
        core_axis_name="c", subcore_axis_name="s",
        num_cores=NC, num_subcores=NS,
    )

    @functools.partial(
        pl.kernel,
        out_type=jax.ShapeDtypeStruct((S, 8, NBT, 8 * CHUNK), jnp.float32),
        mesh=mesh,
        scratch_types=[
            pltpu.VMEM((2, NB, CHUNK), jnp.int32),
            pltpu.VMEM((2, NB, CHUNK), jnp.int32),
            pltpu.VMEM((NB, CHUNK, 128), jnp.float32),
            pltpu.VMEM((NB, 8 * 8 * CHUNK), jnp.float32),
            pltpu.SemaphoreType.DMA,
            pltpu.SemaphoreType.DMA,
            pltpu.SemaphoreType.DMA,
        ],
        compiler_params=pltpu.CompilerParams(
            use_tc_tiling_on_sc=False, needs_layout_passes=False),
    )
    def sc_kernel(idx_hbm, h_hbm, table_hbm, out_hbm, idx_v, h_v,
                  rows_v, stage_v, isem, gsem, ssem):
        wid = lax.axis_index("s") * NC + lax.axis_index("c")
        c0 = wid * n_chunks

        def idx_fetch(g, p):
            pltpu.async_copy(
                idx_hbm.at[pl.ds(c0 + g * NB, NB)], idx_v.at[p], isem)
            pltpu.async_copy(
                h_hbm.at[pl.ds(c0 + g * NB, NB)], h_v.at[p], isem)

        def extract(q, b):
            lane = lax.iota(jnp.int32, 16)

            @plsc.parallel_loop(0, CHUNK // 16)
            def body_tb(tb):
                r0 = 16 * tb
                row_idx = r0 + lane
                h64 = h_v[q, b, pl.ds(r0, 16)]
                for db in range(4):
                    for jg in range(2):
                        vs = []
                        for j8 in range(8):
                            j = 8 * jg + j8
                            skew = (lane + j) & 15
                            d_vec = 16 * db + skew
                            vs.append((d_vec, plsc.load_gather(
                                rows_v.at[b], [row_idx, d_vec + h64])))
                        for d_vec, v in vs:
                            plsc.store_scatter(
                                stage_v.at[b],
                                [d_vec * CHUNK + row_idx], v)

        def drain_stores():
            for _ in range(NB * 8):
                pltpu.make_async_copy(
                    stage_v.at[0, pl.ds(0, CHUNK * 8)],
                    out_hbm.at[0, 0, 0], ssem).wait()

        def one_pass(g2, q):
            g = 2 * g2 + q
            pltpu.make_async_copy(
                idx_hbm.at[pl.ds(0, NB)], idx_v.at[q], isem).wait()
            pltpu.make_async_copy(
                h_hbm.at[pl.ds(0, NB)], h_v.at[q], isem).wait()

            @pl.when(g >= 1)
            def _():
                drain_stores()

            for b in range(NB):
                pltpu.async_copy(
                    table_hbm.at[idx_v.at[q, b]], rows_v.at[b], gsem)

            @pl.when(g + 1 < n_pass)
            def _():
                idx_fetch(g + 1, 1 - q)

            for b in range(NB):
                pltpu.make_async_copy(
                    table_hbm.at[idx_v.at[q, b]], rows_v.at[b], gsem).wait()
                extract(q, b)
                c = c0 + g * NB + b
                s = c // NBT
                bt = lax.rem(c, NBT)
                for dt in range(8):
                    pltpu.async_copy(
                        stage_v.at[b, pl.ds(dt * CHUNK * 8, CHUNK * 8)],
                        out_hbm.at[s, dt, bt], ssem)

        idx_fetch(0, 0)

        def outer(g2, carry):
            one_pass(g2, 0)
            one_pass(g2, 1)
            return carry

        lax.fori_loop(0, n_pass // 2, outer, 0)
        drain_stores()

    return sc_kernel


def kernel(token_ids, weight):
    B0, S = token_ids.shape
    V, D = weight.shape
    B = B0 * S
    info = plsc.get_sparse_core_info()
    wt = jnp.transpose(weight)
    table = _build_tc(V, D)(wt, wt)
    x = jnp.transpose(token_ids).reshape(B // CHUNK, CHUNK).astype(jnp.int32)
    half = 62 * CB
    h = (x >= half).astype(jnp.int32)
    row2d = x - h * half
    h2d = h << 6
    z = _build_sc(B0, S, V, info.num_cores, info.num_subcores)(
        row2d, h2d, table)
    z = z.reshape(S, 8, B0 // CHUNK, 8, CHUNK)
    return z.transpose(2, 4, 0, 1, 3).reshape(B0, S, D)
